# baseline (speedup 1.0000x reference)
import functools

import jax
import jax.numpy as jnp
from jax import lax
from jax.experimental import pallas as pl
from jax.experimental.pallas import tpu as pltpu
from jax.experimental.pallas import tpu_sc as plsc


_NUM_CORES = 2
_NUM_SUBCORES = 16
_NUM_WORKERS = _NUM_CORES * _NUM_SUBCORES
_CHUNK_ROWS = 32


def _sc_add_body(x_hbm, emb_hbm, out_hbm, xv, ev):
    batch, seq_len, dim = x_hbm.shape
    wid = lax.axis_index("s") * _NUM_CORES + lax.axis_index("c")
    rows_per_worker = seq_len // _NUM_WORKERS
    n_chunks = rows_per_worker // _CHUNK_ROWS
    vecs_per_chunk = _CHUNK_ROWS * dim // 16

    def chunk_body(c, carry):
        base = wid * rows_per_worker + c * _CHUNK_ROWS
        pltpu.sync_copy(emb_hbm.at[pl.ds(base, _CHUNK_ROWS)], ev)
        for b in range(batch):
            pltpu.sync_copy(x_hbm.at[b, pl.ds(base, _CHUNK_ROWS)], xv)

            def vec_body(i, carry2):
                r = i // (dim // 16)
                off = (i % (dim // 16)) * 16
                xv[r, pl.ds(off, 16)] = xv[r, pl.ds(off, 16)] + ev[r, pl.ds(off, 16)]
                return carry2

            lax.fori_loop(0, vecs_per_chunk, vec_body, 0)
            pltpu.sync_copy(xv, out_hbm.at[b, pl.ds(base, _CHUNK_ROWS)])
        return carry

    lax.fori_loop(0, n_chunks, chunk_body, 0)


def kernel(x, emb_weight):
    batch, seq_len, dim = x.shape
    mesh = plsc.VectorSubcoreMesh(core_axis_name="c", subcore_axis_name="s")
    sc_call = functools.partial(
        pl.kernel,
        mesh=mesh,
        out_type=jax.ShapeDtypeStruct(x.shape, x.dtype),
        scratch_types=[
            pltpu.VMEM((_CHUNK_ROWS, dim), jnp.float32),
            pltpu.VMEM((_CHUNK_ROWS, dim), jnp.float32),
        ],
    )(_sc_add_body)
    return sc_call(x, emb_weight)


# speedup vs baseline: 1.7161x; 1.7161x over previous
import functools

import jax
import jax.numpy as jnp
from jax import lax
from jax.experimental import pallas as pl
from jax.experimental.pallas import tpu as pltpu
from jax.experimental.pallas import tpu_sc as plsc


_NUM_CORES = 2
_NUM_SUBCORES = 16
_NUM_WORKERS = _NUM_CORES * _NUM_SUBCORES
_CHUNK_ROWS = 32


def _sc_add_body(x_hbm, emb_hbm, out_hbm, xv, ev):
    batch, seq_len, dim = x_hbm.shape
    wid = lax.axis_index("s") * _NUM_CORES + lax.axis_index("c")
    rows_per_worker = seq_len // _NUM_WORKERS
    n_chunks = rows_per_worker // _CHUNK_ROWS
    vecs_per_chunk = _CHUNK_ROWS * dim // 16

    def chunk_body(c, carry):
        base = wid * rows_per_worker + c * _CHUNK_ROWS
        pltpu.sync_copy(emb_hbm.at[pl.ds(base, _CHUNK_ROWS)], ev)
        for b in range(batch):
            pltpu.sync_copy(x_hbm.at[b, pl.ds(base, _CHUNK_ROWS)], xv)

            def row_body(r, carry2):
                for j in range(dim // 16):
                    off = j * 16
                    xv[r, pl.ds(off, 16)] = xv[r, pl.ds(off, 16)] + ev[r, pl.ds(off, 16)]
                return carry2

            lax.fori_loop(0, _CHUNK_ROWS, row_body, 0)
            pltpu.sync_copy(xv, out_hbm.at[b, pl.ds(base, _CHUNK_ROWS)])
        return carry

    lax.fori_loop(0, n_chunks, chunk_body, 0)


def kernel(x, emb_weight):
    batch, seq_len, dim = x.shape
    mesh = plsc.VectorSubcoreMesh(core_axis_name="c", subcore_axis_name="s")
    sc_call = functools.partial(
        pl.kernel,
        mesh=mesh,
        out_type=jax.ShapeDtypeStruct(x.shape, x.dtype),
        scratch_types=[
            pltpu.VMEM((_CHUNK_ROWS, dim), jnp.float32),
            pltpu.VMEM((_CHUNK_ROWS, dim), jnp.float32),
        ],
    )(_sc_add_body)
    return sc_call(x, emb_weight)


# speedup vs baseline: 4.7594x; 2.7733x over previous
import jax
import jax.numpy as jnp
from jax.experimental import pallas as pl


_SEQ_BLOCK = 512


def _add_kernel(x_ref, emb_ref, out_ref):
    out_ref[...] = x_ref[...] + emb_ref[...][None, :, :]


def kernel(x, emb_weight):
    batch, seq_len, dim = x.shape
    grid = (seq_len // _SEQ_BLOCK,)
    return pl.pallas_call(
        _add_kernel,
        grid=grid,
        in_specs=[
            pl.BlockSpec((batch, _SEQ_BLOCK, dim), lambda i: (0, i, 0)),
            pl.BlockSpec((_SEQ_BLOCK, dim), lambda i: (i, 0)),
        ],
        out_specs=pl.BlockSpec((batch, _SEQ_BLOCK, dim), lambda i: (0, i, 0)),
        out_shape=jax.ShapeDtypeStruct(x.shape, x.dtype),
    )(x, emb_weight)
